# baseline (speedup 1.0000x reference)
import jax
import jax.numpy as jnp
from jax.experimental import pallas as pl
from jax.experimental.pallas import tpu as pltpu
from jax.experimental.pallas import tpu_sc as plsc

_SC_CORES = 2
_SC_CHUNK_ROWS = 512


def kernel(inputs, embeddings):
    seq_len = inputs.shape[1]
    emb_dim = embeddings.shape[1]
    table = embeddings[:seq_len, :]
    rows_per_core = seq_len // _SC_CORES
    ch = min(_SC_CHUNK_ROWS, rows_per_core)
    nch = rows_per_core // ch
    mesh = plsc.ScalarSubcoreMesh(axis_name="core", num_cores=_SC_CORES)

    @pl.kernel(
        out_type=jax.ShapeDtypeStruct((seq_len, emb_dim), table.dtype),
        mesh=mesh,
        scratch_types=[
            pltpu.VMEM_SHARED((2, ch, emb_dim), table.dtype),
            pltpu.SemaphoreType.DMA((2,)),
            pltpu.SemaphoreType.DMA((2,)),
        ],
    )
    def sc_copy(x_hbm, o_hbm, buf, isem, osem):
        core = jax.lax.axis_index("core")
        base = core * rows_per_core

        def in_copy(k, slot):
            return pltpu.make_async_copy(
                x_hbm.at[pl.ds(base + k * ch, ch)], buf.at[slot],
                isem.at[slot])

        def out_copy(k, slot):
            return pltpu.make_async_copy(
                buf.at[slot], o_hbm.at[pl.ds(base + k * ch, ch)],
                osem.at[slot])

        for s in range(min(2, nch)):
            in_copy(s, s).start()
        for k in range(nch):
            slot = k % 2
            in_copy(k, slot).wait()
            out_copy(k, slot).start()
            if k + 2 < nch:
                out_copy(k, slot).wait()
                in_copy(k + 2, slot).start()
        for k in range(max(0, nch - 2), nch):
            out_copy(k, k % 2).wait()

    return sc_copy(table)[None]


# speedup vs baseline: 1.0865x; 1.0865x over previous
import jax
import jax.numpy as jnp
from jax.experimental import pallas as pl
from jax.experimental.pallas import tpu as pltpu
from jax.experimental.pallas import tpu_sc as plsc

_SC_CORES = 2
_SC_SUBCORES = 16
_SC_CHUNK_ROWS = 32


def kernel(inputs, embeddings):
    seq_len = inputs.shape[1]
    emb_dim = embeddings.shape[1]
    table = embeddings[:seq_len, :]
    workers = _SC_CORES * _SC_SUBCORES
    rows_per_tec = seq_len // workers
    ch = min(_SC_CHUNK_ROWS, rows_per_tec)
    nch = rows_per_tec // ch
    mesh = plsc.VectorSubcoreMesh(
        core_axis_name="core", subcore_axis_name="subcore")

    @pl.kernel(
        out_type=jax.ShapeDtypeStruct((seq_len, emb_dim), table.dtype),
        mesh=mesh,
        scratch_types=[
            pltpu.VMEM((2, ch, emb_dim), table.dtype),
            pltpu.SemaphoreType.DMA((2,)),
            pltpu.SemaphoreType.DMA((2,)),
        ],
    )
    def sc_copy(x_hbm, o_hbm, buf, isem, osem):
        core = jax.lax.axis_index("core")
        sub = jax.lax.axis_index("subcore")
        base = (core * _SC_SUBCORES + sub) * rows_per_tec

        def in_copy(k, slot):
            return pltpu.make_async_copy(
                x_hbm.at[pl.ds(base + k * ch, ch)], buf.at[slot],
                isem.at[slot])

        def out_copy(k, slot):
            return pltpu.make_async_copy(
                buf.at[slot], o_hbm.at[pl.ds(base + k * ch, ch)],
                osem.at[slot])

        for s in range(min(2, nch)):
            in_copy(s, s).start()
        for k in range(nch):
            slot = k % 2
            in_copy(k, slot).wait()
            out_copy(k, slot).start()
            if k + 2 < nch:
                out_copy(k, slot).wait()
                in_copy(k + 2, slot).start()
        for k in range(max(0, nch - 2), nch):
            out_copy(k, k % 2).wait()

    return sc_copy(table)[None]


# speedup vs baseline: 1.1037x; 1.0158x over previous
import jax
import jax.numpy as jnp
from jax.experimental import pallas as pl
from jax.experimental.pallas import tpu as pltpu
from jax.experimental.pallas import tpu_sc as plsc

_SC_CORES = 2
_SC_SUBCORES = 16
_SC_CHUNK_ROWS = 32
_SC_NBUF = 3


def kernel(inputs, embeddings):
    seq_len = inputs.shape[1]
    emb_dim = embeddings.shape[1]
    table = embeddings[:seq_len, :]
    workers = _SC_CORES * _SC_SUBCORES
    rows_per_tec = seq_len // workers
    ch = min(_SC_CHUNK_ROWS, rows_per_tec)
    nch = rows_per_tec // ch
    mesh = plsc.VectorSubcoreMesh(
        core_axis_name="core", subcore_axis_name="subcore")

    @pl.kernel(
        out_type=jax.ShapeDtypeStruct((seq_len, emb_dim), table.dtype),
        mesh=mesh,
        scratch_types=[
            pltpu.VMEM((_SC_NBUF, ch, emb_dim), table.dtype),
            pltpu.SemaphoreType.DMA((_SC_NBUF,)),
            pltpu.SemaphoreType.DMA((_SC_NBUF,)),
        ],
    )
    def sc_copy(x_hbm, o_hbm, buf, isem, osem):
        core = jax.lax.axis_index("core")
        sub = jax.lax.axis_index("subcore")
        base = (core * _SC_SUBCORES + sub) * rows_per_tec

        def in_copy(k, slot):
            return pltpu.make_async_copy(
                x_hbm.at[pl.ds(base + k * ch, ch)], buf.at[slot],
                isem.at[slot])

        def out_copy(k, slot):
            return pltpu.make_async_copy(
                buf.at[slot], o_hbm.at[pl.ds(base + k * ch, ch)],
                osem.at[slot])

        nbuf = min(_SC_NBUF, nch)
        for s in range(nbuf):
            in_copy(s, s).start()
        for k in range(nch):
            slot = k % nbuf
            in_copy(k, slot).wait()
            out_copy(k, slot).start()
            if k + nbuf < nch:
                out_copy(k, slot).wait()
                in_copy(k + nbuf, slot).start()
        for k in range(max(0, nch - nbuf), nch):
            out_copy(k, k % nbuf).wait()

    return sc_copy(table)[None]


# speedup vs baseline: 3.1740x; 2.8759x over previous
import jax
import jax.numpy as jnp
from jax.experimental import pallas as pl
from jax.experimental.pallas import tpu as pltpu

_BLOCK_ROWS = 2048


def _copy_block(emb_ref, out_ref):
    out_ref[...] = emb_ref[...]


def kernel(inputs, embeddings):
    seq_len = inputs.shape[1]
    emb_dim = embeddings.shape[1]
    table = embeddings[:seq_len, :]
    blk = min(_BLOCK_ROWS, seq_len)
    grid = (seq_len // blk,)
    out = pl.pallas_call(
        _copy_block,
        grid=grid,
        in_specs=[pl.BlockSpec((blk, emb_dim), lambda i: (i, 0))],
        out_specs=pl.BlockSpec((blk, emb_dim), lambda i: (i, 0)),
        out_shape=jax.ShapeDtypeStruct((seq_len, emb_dim), embeddings.dtype),
        compiler_params=pltpu.CompilerParams(
            dimension_semantics=("arbitrary",),
        ),
    )(table)
    return out[None]
